# baseline (speedup 1.0000x reference)
import functools

import jax
import jax.numpy as jnp
from jax import lax
from jax.experimental import pallas as pl
from jax.experimental.pallas import tpu as pltpu
from jax.experimental.pallas import tpu_sc as plsc

NUM_EXPERTS = 8
TOP_K = 2
HIDDEN = 1024
INTER = 4096
NUM_TOKENS = 2048

TB = 256
IB = 1024
NB = (NUM_TOKENS * TOP_K) // TB + NUM_EXPERTS
P = NB * TB
NW = 32
DISPATCH_CHUNK = 32
COMBINE_CHUNK = 32


def _routing_kernel(x_ref, wg_ref, eidx_ref, ew_ref):
    logits = jnp.dot(x_ref[...], wg_ref[...],
                     preferred_element_type=jnp.float32)
    lanes = lax.broadcasted_iota(jnp.int32, logits.shape, 1)
    big = jnp.int32(NUM_EXPERTS)
    m1 = jnp.max(logits, axis=1, keepdims=True)
    i1 = jnp.min(jnp.where(logits == m1, lanes, big), axis=1, keepdims=True)
    l2 = jnp.where(lanes == i1, -jnp.inf, logits)
    m2 = jnp.max(l2, axis=1, keepdims=True)
    i2 = jnp.min(jnp.where(l2 == m2, lanes, big), axis=1, keepdims=True)
    klane = lax.broadcasted_iota(jnp.int32, eidx_ref.shape, 1)
    eidx_ref[...] = jnp.where(klane == 0, i1, i2)
    w1 = 1.0 / (1.0 + jnp.exp(m2 - m1))
    ew_ref[...] = jnp.where(klane == 0, w1, 1.0 - w1)


def _routing(hidden_states, w_gate):
    return pl.pallas_call(
        _routing_kernel,
        grid=(NUM_TOKENS // TB,),
        in_specs=[
            pl.BlockSpec((TB, HIDDEN), lambda t: (t, 0)),
            pl.BlockSpec((HIDDEN, NUM_EXPERTS), lambda t: (0, 0)),
        ],
        out_specs=[
            pl.BlockSpec((TB, TOP_K), lambda t: (t, 0)),
            pl.BlockSpec((TB, TOP_K), lambda t: (t, 0)),
        ],
        out_shape=[
            jax.ShapeDtypeStruct((NUM_TOKENS, TOP_K), jnp.int32),
            jax.ShapeDtypeStruct((NUM_TOKENS, TOP_K), jnp.float32),
        ],
    )(hidden_states, w_gate)


def _dispatch_metadata(eidx, ew):
    e_flat = eidx.reshape(-1)
    w_flat = ew.reshape(-1)
    onehot = (e_flat[:, None] == jnp.arange(NUM_EXPERTS)[None, :]).astype(
        jnp.int32)
    ranks = jnp.cumsum(onehot, axis=0)
    counts = ranks[-1]
    rank = jnp.take_along_axis(ranks, e_flat[:, None], axis=1)[:, 0] - 1
    blocks_per_e = (counts + TB - 1) // TB
    bcum = jnp.cumsum(blocks_per_e)
    bstart = bcum - blocks_per_e
    pos = (bstart[e_flat] * TB + rank).astype(jnp.int32)
    block_expert = jnp.searchsorted(
        bcum, jnp.arange(NB, dtype=jnp.int32), side="right").astype(jnp.int32)
    block_expert = jnp.minimum(block_expert, NUM_EXPERTS - 1)
    pos2 = pos.reshape(NUM_TOKENS, TOP_K)
    tok_w = NUM_TOKENS // NW
    pos3 = pos2.reshape(NW, tok_w, TOP_K).transpose(0, 2, 1).reshape(
        NW * TOP_K, tok_w)
    comb_idx = jnp.concatenate([pos2[:, 0], pos2[:, 1]]).astype(jnp.int32)
    comb_w = jnp.broadcast_to(
        jnp.concatenate([ew[:, 0], ew[:, 1]]).astype(jnp.float32)[:, None],
        (NUM_TOKENS * TOP_K, 16))
    nb_used = bcum[-1].astype(jnp.int32)
    return pos3, block_expert, comb_idx, comb_w, nb_used


@functools.cache
def _dispatch_kernel():
    mesh = plsc.VectorSubcoreMesh(core_axis_name="c", subcore_axis_name="s")
    tok_w = NUM_TOKENS // NW

    @functools.partial(
        pl.kernel,
        mesh=mesh,
        out_type=jax.ShapeDtypeStruct((P, HIDDEN), jnp.float32),
        scratch_types=[
            pltpu.VMEM((tok_w,), jnp.int32),
            pltpu.VMEM((tok_w,), jnp.int32),
            pltpu.VMEM((tok_w, HIDDEN), jnp.float32),
            pltpu.SemaphoreType.DMA,
            pltpu.SemaphoreType.DMA,
        ],
    )
    def dispatch(x_hbm, idx_hbm, out_hbm, ia_v, ib_v, xbuf, s0, s1):
        wid = lax.axis_index("s") * 2 + lax.axis_index("c")
        pltpu.sync_copy(idx_hbm.at[wid * TOP_K], ia_v)
        pltpu.sync_copy(idx_hbm.at[wid * TOP_K + 1], ib_v)
        pltpu.sync_copy(x_hbm.at[pl.ds(wid * tok_w, tok_w)], xbuf)
        c0 = pltpu.async_copy(xbuf, out_hbm.at[ia_v], s0)
        c1 = pltpu.async_copy(xbuf, out_hbm.at[ib_v], s1)
        c0.wait()
        c1.wait()

    return dispatch


def _dispatch_call(x, pos3):
    return _dispatch_kernel()(x, pos3)


def _ffn_kernel(be_ref, x_ref, w1_ref, w3_ref, w2_ref, y_ref):
    ib = pl.program_id(1)
    x = x_ref[...]
    h = jnp.dot(x, w1_ref[0], preferred_element_type=jnp.float32)
    g = jnp.dot(x, w3_ref[0], preferred_element_type=jnp.float32)
    act = (h / (1.0 + jnp.exp(-h))) * g
    y = jnp.dot(act, w2_ref[0], preferred_element_type=jnp.float32)

    @pl.when(ib == 0)
    def _():
        y_ref[...] = y

    @pl.when(ib > 0)
    def _():
        y_ref[...] += y


def _ffn(block_expert, x_sorted, w1, w3, w2, nb_used):
    grid_spec = pltpu.PrefetchScalarGridSpec(
        num_scalar_prefetch=1,
        grid=(nb_used, INTER // IB),
        in_specs=[
            pl.BlockSpec((TB, HIDDEN), lambda b, ib, be: (b, 0)),
            pl.BlockSpec((1, HIDDEN, IB), lambda b, ib, be: (be[b], 0, ib)),
            pl.BlockSpec((1, HIDDEN, IB), lambda b, ib, be: (be[b], 0, ib)),
            pl.BlockSpec((1, IB, HIDDEN), lambda b, ib, be: (be[b], ib, 0)),
        ],
        out_specs=pl.BlockSpec((TB, HIDDEN), lambda b, ib, be: (b, 0)),
    )
    return pl.pallas_call(
        _ffn_kernel,
        grid_spec=grid_spec,
        out_shape=jax.ShapeDtypeStruct((P, HIDDEN), jnp.float32),
        compiler_params=pltpu.CompilerParams(
            dimension_semantics=("arbitrary", "arbitrary"),
        ),
    )(block_expert, x_sorted, w1, w3, w2)


@functools.cache
def _combine_kernel():
    mesh = plsc.VectorSubcoreMesh(core_axis_name="c", subcore_axis_name="s")

    @functools.partial(
        pl.kernel,
        mesh=mesh,
        out_type=jax.ShapeDtypeStruct((NUM_TOKENS, HIDDEN), jnp.float32),
        scratch_types=[
            pltpu.VMEM((COMBINE_CHUNK,), jnp.int32),
            pltpu.VMEM((COMBINE_CHUNK,), jnp.int32),
            pltpu.VMEM((COMBINE_CHUNK, 16), jnp.float32),
            pltpu.VMEM((COMBINE_CHUNK, 16), jnp.float32),
            pltpu.VMEM((COMBINE_CHUNK, HIDDEN), jnp.float32),
            pltpu.VMEM((COMBINE_CHUNK, HIDDEN), jnp.float32),
            pltpu.SemaphoreType.DMA,
            pltpu.SemaphoreType.DMA,
        ],
    )
    def combine(y_hbm, idx_hbm, w_hbm, out_hbm,
                ia_v, ib_v, wa_v, wb_v, a_v, b_v, sa, sb):
        wid = lax.axis_index("s") * 2 + lax.axis_index("c")
        tok_w = NUM_TOKENS // NW
        base = wid * tok_w
        for c in range(tok_w // COMBINE_CHUNK):
            off = base + c * COMBINE_CHUNK
            pltpu.sync_copy(idx_hbm.at[pl.ds(off, COMBINE_CHUNK)], ia_v)
            pltpu.sync_copy(
                idx_hbm.at[pl.ds(NUM_TOKENS + off, COMBINE_CHUNK)], ib_v)
            pltpu.sync_copy(w_hbm.at[pl.ds(off, COMBINE_CHUNK), :], wa_v)
            pltpu.sync_copy(
                w_hbm.at[pl.ds(NUM_TOKENS + off, COMBINE_CHUNK), :], wb_v)
            ca = pltpu.async_copy(y_hbm.at[ia_v], a_v, sa)
            cb = pltpu.async_copy(y_hbm.at[ib_v], b_v, sb)
            ca.wait()
            cb.wait()

            def body(r, carry):
                wa = wa_v[r, :]
                wb = wb_v[r, :]
                for j in range(HIDDEN // 16):
                    sl = pl.ds(j * 16, 16)
                    a_v[r, sl] = a_v[r, sl] * wa + b_v[r, sl] * wb
                return carry

            lax.fori_loop(0, COMBINE_CHUNK, body, 0)
            pltpu.sync_copy(a_v, out_hbm.at[pl.ds(off, COMBINE_CHUNK)])

    return combine


def _combine_call(y_sorted, comb_idx, comb_w):
    return _combine_kernel()(y_sorted, comb_idx, comb_w)


@jax.jit
def kernel(hidden_states, w_gate, w1, w2, w3):
    eidx, ew = _routing(hidden_states, w_gate)
    pos3, block_expert, comb_idx, comb_w, nb_used = (
        _dispatch_metadata(eidx, ew))
    x_sorted = _dispatch_call(hidden_states, pos3)
    y_sorted = _ffn(block_expert, x_sorted, w1, w3, w2, nb_used)
    return _combine_call(y_sorted, comb_idx, comb_w)
